# Initial kernel scaffold; baseline (speedup 1.0000x reference)
import jax
import jax.numpy as jnp
from jax.experimental import pallas as pl


def kernel(layout, mask_agent_ijx):
    raise NotImplementedError("write your pallas kernel here")


# speedup vs baseline: 2.4360x; 2.4360x over previous
import jax
import jax.numpy as jnp
from jax.experimental import pallas as pl


def _body(msk_ref, lay_ref, out_ref):
    m = msk_ref[...]
    bb, mwidth = m.shape
    iota = jax.lax.broadcasted_iota(jnp.int32, (bb, mwidth), 1)
    pos = jnp.sum(iota * m.astype(jnp.int32), axis=1, keepdims=True)
    x_agent = pos & 3
    cell = pos >> 2
    clear_sword = (x_agent & 1) == 0
    clear_shield = (x_agent >> 1) == 0

    v = lay_ref[...]
    is_sword = v == 11
    is_shield = v == 12
    ch1 = jnp.where(v == 9, 4, 0)
    ch1 = jnp.where(is_sword, 3, ch1)
    ch1 = jnp.where(is_shield, 2, ch1)
    ch1 = jnp.where(v == 13, 1, ch1)
    ch0 = jnp.where(is_sword & clear_sword, 1, v)
    ch0 = jnp.where(is_shield & clear_shield, 1, ch0)
    hw = v.shape[1]
    cell_iota = jax.lax.broadcasted_iota(jnp.int32, (bb, hw), 1)
    ch0 = jnp.where(cell_iota == cell, 10, ch0)
    out_ref[...] = (ch0 | (ch1 << 8)).astype(jnp.uint16)


def kernel(layout, mask_agent_ijx):
    b, h, w, _ = layout.shape
    hw = h * w
    lay = layout.reshape(b, hw)
    msk = mask_agent_ijx.view(jnp.int8)
    bb = 256
    grid = (b // bb,)
    out16 = pl.pallas_call(
        _body,
        grid=grid,
        in_specs=[
            pl.BlockSpec((bb, 4 * hw), lambda i: (i, 0)),
            pl.BlockSpec((bb, hw), lambda i: (i, 0)),
        ],
        out_specs=pl.BlockSpec((bb, hw), lambda i: (i, 0)),
        out_shape=jax.ShapeDtypeStruct((b, hw), jnp.uint16),
    )(msk, lay)
    out8 = jax.lax.bitcast_convert_type(out16, jnp.uint8)
    return out8.reshape(b, h, w, 2)
